# baseline (speedup 1.0000x reference)
import functools

import jax
import jax.numpy as jnp
from jax import lax
from jax.experimental import pallas as pl
from jax.experimental.pallas import tpu as pltpu
from jax.experimental.pallas import tpu_sc as plsc

N = 10000
E = 320000
D = 128
H = 128
G = 16
HH = H // 2

NC = 2
NS = 16
NW = NC * NS

ZCHUNK = 80
RCHUNKS = N // ZCHUNK

DCH = 96
EPW = E // NW
DNF = EPW // DCH
DTL = EPW - DNF * DCH

ECH = 128
EPT = E // NS
ENF = EPT // ECH
ETL = EPT - ENF * ECH
EPH = ENF // 2
NBUF = 6
ENR = EPH // NBUF
SH0 = EPH * ECH
SH1 = EPT - SH0

_SC_MESH = plsc.VectorSubcoreMesh(
    core_axis_name="c", subcore_axis_name="s", num_cores=NC, num_subcores=NS)


def _sc_degree(dstm, dstt):
    @functools.partial(
        pl.kernel,
        out_type=jax.ShapeDtypeStruct((NC * N,), jnp.float32),
        mesh=_SC_MESH,
        scratch_types=[
            pltpu.VMEM((DNF, DCH), jnp.int32),
            pltpu.VMEM((DTL,), jnp.int32),
            pltpu.VMEM((DCH,), jnp.float32),
            pltpu.VMEM((ZCHUNK,), jnp.float32),
            pltpu.VMEM_SHARED((N,), jnp.float32),
            pltpu.SemaphoreType.DMA,
        ],
    )
    def deg_kernel(dstm_hbm, dstt_hbm, out_hbm, dstidx, dstT, valbuf, zbuf,
                   acc, sem):
        c = lax.axis_index("c")
        s = lax.axis_index("s")
        wid = c * NS + s

        ones16 = jnp.ones((16,), jnp.float32)
        zero16 = jnp.zeros((16,), jnp.float32)

        def fill(i, _):
            valbuf[pl.ds(i * 16, 16)] = ones16
            return 0
        lax.fori_loop(0, DCH // 16, fill, 0)

        def zfill(i, _):
            zbuf[pl.ds(i * 16, 16)] = zero16
            return 0
        lax.fori_loop(0, ZCHUNK // 16, zfill, 0)

        pltpu.sync_copy(dstm_hbm.at[wid], dstidx)
        pltpu.sync_copy(dstt_hbm.at[wid], dstT)

        def acc_zero(j, _):
            k = s * 8 + j

            @pl.when(k < RCHUNKS)
            def _():
                pltpu.sync_copy(zbuf, acc.at[pl.ds(k * ZCHUNK, ZCHUNK)])
            return 0
        lax.fori_loop(0, 8, acc_zero, 0)
        plsc.subcore_barrier()

        def fire(g, _):
            pltpu.async_copy(valbuf, acc.at[dstidx.at[g]], sem, add=True)
            return 0
        lax.fori_loop(0, DNF, fire, 0)

        def drain(g, _):
            pltpu.make_async_copy(valbuf, acc.at[dstidx.at[0]], sem).wait()
            return 0
        lax.fori_loop(0, DNF, drain, 0)

        pltpu.sync_copy(valbuf.at[pl.ds(0, DTL)], acc.at[dstT], add=True)
        plsc.subcore_barrier()

        obase = c * N

        def dump(j, _):
            k = s * 8 + j

            @pl.when(k < RCHUNKS)
            def _():
                pltpu.sync_copy(acc.at[pl.ds(k * ZCHUNK, ZCHUNK)], zbuf)
                pltpu.sync_copy(zbuf, out_hbm.at[pl.ds(obase + k * ZCHUNK, ZCHUNK)])
            return 0
        lax.fori_loop(0, 8, dump, 0)

    return deg_kernel(dstm, dstt)


def _sc_scatter(tableF, srcf, dstmA, dstmB, dstt):
    @functools.partial(
        pl.kernel,
        out_type=jax.ShapeDtypeStruct((NC, N, HH), jnp.float32),
        mesh=_SC_MESH,
        scratch_types=[
            pltpu.VMEM((SH1,), jnp.int32),
            pltpu.VMEM((EPH, ECH), jnp.int32),
            pltpu.VMEM((ETL,), jnp.int32),
            [pltpu.VMEM((ECH, HH), jnp.float32)] * NBUF,
            pltpu.VMEM_SHARED((N, HH), jnp.float32),
            [pltpu.SemaphoreType.DMA] * NBUF,
            [pltpu.SemaphoreType.DMA] * NBUF,
        ],
        compiler_params=pltpu.CompilerParams(use_tc_tiling_on_sc=False),
    )
    def scat_kernel(table_hbm, src_hbm, dstmA_hbm, dstmB_hbm, dstt_hbm,
                    out_hbm, srcidx, dstidx, dstT, rows, acc, gsems, ssems):
        c = lax.axis_index("c")
        s = lax.axis_index("s")
        cN = c * N

        zero16 = jnp.zeros((16,), jnp.float32)

        def load_src_half(hbm_off, nwords):
            pltpu.sync_copy(src_hbm.at[pl.ds(hbm_off, nwords)],
                            srcidx.at[pl.ds(0, nwords)])

            def adj(i, _):
                sl = pl.ds(i * 16, 16)
                srcidx[sl] = srcidx[sl] + cN
                return 0
            lax.fori_loop(0, nwords // 16, adj, 0)

        base = s * EPT
        load_src_half(base, SH0)
        pltpu.sync_copy(dstmA_hbm.at[s], dstidx)
        pltpu.sync_copy(dstt_hbm.at[s], dstT)

        def zrow(r, _):
            def zcol(cc, _):
                rows[0][r, pl.ds(cc * 16, 16)] = zero16
                return 0
            lax.fori_loop(0, HH // 16, zcol, 0)
            return 0
        lax.fori_loop(0, ZCHUNK, zrow, 0)

        def acc_zero(j, _):
            k = s * 8 + j

            @pl.when(k < RCHUNKS)
            def _():
                pltpu.sync_copy(rows[0].at[pl.ds(0, ZCHUNK), :],
                                acc.at[pl.ds(k * ZCHUNK, ZCHUNK), :])
            return 0
        lax.fori_loop(0, 8, acc_zero, 0)
        plsc.subcore_barrier()

        def gather(l, b):
            pltpu.async_copy(table_hbm.at[srcidx.at[pl.ds(l * ECH, ECH)]],
                             rows[b], gsems[b])

        def gather_wait(b):
            pltpu.make_async_copy(
                table_hbm.at[srcidx.at[pl.ds(0, ECH)]], rows[b],
                gsems[b]).wait()

        def scatter(l, b):
            pltpu.async_copy(rows[b], acc.at[dstidx.at[l]], ssems[b],
                             add=True)

        def scatter_wait(b):
            pltpu.make_async_copy(rows[b], acc.at[dstidx.at[0]],
                                  ssems[b]).wait()

        def round_body(r, _):
            for b in range(NBUF):
                gather_wait(b)
                scatter(NBUF * r + b, b)
            for b in range(NBUF):
                scatter_wait(b)
                l = NBUF * r + NBUF + b

                @pl.when(l < EPH)
                def _():
                    gather(l, b)
            return 0

        for b in range(NBUF):
            gather(b, b)
        lax.fori_loop(0, ENR, round_body, 0)

        load_src_half(base + SH0, SH1)
        pltpu.sync_copy(dstmB_hbm.at[s], dstidx)

        for b in range(NBUF):
            gather(b, b)
        lax.fori_loop(0, ENR, round_body, 0)

        pltpu.sync_copy(
            table_hbm.at[srcidx.at[pl.ds(EPH * ECH, ETL)]],
            rows[0].at[pl.ds(0, ETL), :])
        pltpu.sync_copy(rows[0].at[pl.ds(0, ETL), :], acc.at[dstT], add=True)
        plsc.subcore_barrier()

        def dump(j, _):
            k = s * 8 + j

            @pl.when(k < RCHUNKS)
            def _():
                pltpu.sync_copy(acc.at[pl.ds(k * ZCHUNK, ZCHUNK), :],
                                rows[0].at[pl.ds(0, ZCHUNK), :])
                pltpu.sync_copy(rows[0].at[pl.ds(0, ZCHUNK), :],
                                out_hbm.at[c, pl.ds(k * ZCHUNK, ZCHUNK), :])
            return 0
        lax.fori_loop(0, 8, dump, 0)

    return scat_kernel(tableF, srcf, dstmA, dstmB, dstt)


_BLK = 1000
_NBLK = N // _BLK


def _tc_prescale(x, W1, degp):
    def body(x_ref, w_ref, dp_ref, hw_ref, dinv_ref):
        deg = 1.0 + dp_ref[0] + dp_ref[1]
        dinv = lax.rsqrt(deg)
        dinv_ref[...] = dinv
        hw = jnp.dot(x_ref[...], w_ref[...],
                     preferred_element_type=jnp.float32) * dinv
        hw_ref[0] = hw[:, :HH]
        hw_ref[1] = hw[:, HH:]

    return pl.pallas_call(
        body,
        grid=(_NBLK,),
        in_specs=[
            pl.BlockSpec((_BLK, D), lambda i: (i, 0)),
            pl.BlockSpec((D, H), lambda i: (0, 0)),
            pl.BlockSpec((NC, _BLK, 1), lambda i: (0, i, 0)),
        ],
        out_specs=[
            pl.BlockSpec((NC, _BLK, HH), lambda i: (0, i, 0)),
            pl.BlockSpec((_BLK, 1), lambda i: (i, 0)),
        ],
        out_shape=[
            jax.ShapeDtypeStruct((NC, N, HH), jnp.float32),
            jax.ShapeDtypeStruct((N, 1), jnp.float32),
        ],
    )(x, W1, degp)


def _tc_layer_mid(Sp, hwp, dinv, b, W2):
    def body(s_ref, hw_ref, dinv_ref, b_ref, w_ref, out_ref):
        dinv = dinv_ref[...]
        h = jnp.concatenate([s_ref[0] + hw_ref[0], s_ref[1] + hw_ref[1]],
                            axis=1)
        h = jnp.maximum(dinv * h + b_ref[...], 0.0)
        hw = jnp.dot(h, w_ref[...],
                     preferred_element_type=jnp.float32) * dinv
        out_ref[0] = hw[:, :HH]
        out_ref[1] = hw[:, HH:]

    return pl.pallas_call(
        body,
        grid=(_NBLK,),
        in_specs=[
            pl.BlockSpec((NC, _BLK, HH), lambda i: (0, i, 0)),
            pl.BlockSpec((NC, _BLK, HH), lambda i: (0, i, 0)),
            pl.BlockSpec((_BLK, 1), lambda i: (i, 0)),
            pl.BlockSpec((1, H), lambda i: (0, 0)),
            pl.BlockSpec((H, H), lambda i: (0, 0)),
        ],
        out_specs=pl.BlockSpec((NC, _BLK, HH), lambda i: (0, i, 0)),
        out_shape=jax.ShapeDtypeStruct((NC, N, HH), jnp.float32),
    )(Sp, hwp, dinv, b, W2)


def _tc_finish_pool(Sp, hwp, dinv, b, batch2d):
    def body(s_ref, hw_ref, dinv_ref, b_ref, bat_ref, out_ref, cnt_ref):
        i = pl.program_id(0)
        dinv = dinv_ref[...]
        h = jnp.concatenate([s_ref[0] + hw_ref[0], s_ref[1] + hw_ref[1]],
                            axis=1)
        h = jnp.maximum(dinv * h + b_ref[...], 0.0)

        gids = lax.broadcasted_iota(jnp.int32, (_BLK, G), 1)
        onehot = (bat_ref[...] == gids).astype(jnp.float32)
        part = lax.dot_general(onehot, h, (((0,), (0,)), ((), ())),
                               preferred_element_type=jnp.float32)
        pcnt = lax.dot_general(onehot, jnp.ones((_BLK, 1), jnp.float32),
                               (((0,), (0,)), ((), ())),
                               preferred_element_type=jnp.float32)

        @pl.when(i == 0)
        def _():
            out_ref[...] = jnp.zeros_like(out_ref)
            cnt_ref[...] = jnp.zeros_like(cnt_ref)

        out_ref[...] += part
        cnt_ref[...] += pcnt

        @pl.when(i == _NBLK - 1)
        def _():
            out_ref[...] = out_ref[...] / jnp.maximum(cnt_ref[...], 1.0)

    return pl.pallas_call(
        body,
        grid=(_NBLK,),
        in_specs=[
            pl.BlockSpec((NC, _BLK, HH), lambda i: (0, i, 0)),
            pl.BlockSpec((NC, _BLK, HH), lambda i: (0, i, 0)),
            pl.BlockSpec((_BLK, 1), lambda i: (i, 0)),
            pl.BlockSpec((1, H), lambda i: (0, 0)),
            pl.BlockSpec((_BLK, 1), lambda i: (i, 0)),
        ],
        out_specs=pl.BlockSpec((G, H), lambda i: (0, 0)),
        out_shape=jax.ShapeDtypeStruct((G, H), jnp.float32),
        scratch_shapes=[pltpu.VMEM((G, 1), jnp.float32)],
    )(Sp, hwp, dinv, b, batch2d)


def kernel(x, edge_index, batch, W1, b1, W2, b2):
    srcf = edge_index[0]
    dstd = edge_index[1].reshape(NW, EPW)
    dstm = dstd[:, :DNF * DCH].reshape(NW, DNF, DCH)
    dstt = dstd[:, DNF * DCH:]

    dste = edge_index[1].reshape(NS, EPT)
    dstm2 = dste[:, :ENF * ECH].reshape(NS, ENF, ECH)
    dstt2 = dste[:, ENF * ECH:]
    dstm2A = dstm2[:, :EPH, :]
    dstm2B = dstm2[:, EPH:, :]

    degp = _sc_degree(dstm, dstt)
    degp3 = degp.reshape(NC, N, 1)

    hw1p, dinv = _tc_prescale(x, W1, degp3)
    S1 = _sc_scatter(hw1p.reshape(NC * N, HH), srcf, dstm2A, dstm2B, dstt2)
    hw2p = _tc_layer_mid(S1, hw1p, dinv, b1.reshape(1, H), W2)
    S2 = _sc_scatter(hw2p.reshape(NC * N, HH), srcf, dstm2A, dstm2B, dstt2)
    g = _tc_finish_pool(S2, hw2p, dinv, b2.reshape(1, H),
                        batch.reshape(N, 1))
    return g


# speedup vs baseline: 1.0717x; 1.0717x over previous
import functools

import jax
import jax.numpy as jnp
from jax import lax
from jax.experimental import pallas as pl
from jax.experimental.pallas import tpu as pltpu
from jax.experimental.pallas import tpu_sc as plsc

N = 10000
E = 320000
D = 128
H = 128
G = 16

NC = 2
NS = 16
NW = NC * NS

CHUNK = 80
EPW = E // NW
NFULL = EPW // CHUNK
NBUF = 3
PH0 = 63
PH1 = NFULL - PH0
NR0 = PH0 // NBUF
NR1 = PH1 // NBUF
NLEFT = PH1 - NR1 * NBUF
ZCHUNK = 80
RCHUNKS = N // ZCHUNK

_SC_MESH = plsc.VectorSubcoreMesh(
    core_axis_name="c", subcore_axis_name="s", num_cores=NC, num_subcores=NS)


def _sc_degree(dstm):
    @functools.partial(
        pl.kernel,
        out_type=jax.ShapeDtypeStruct((NC * N,), jnp.float32),
        mesh=_SC_MESH,
        scratch_types=[
            pltpu.VMEM((NFULL, CHUNK), jnp.int32),
            pltpu.VMEM((CHUNK,), jnp.float32),
            pltpu.VMEM((ZCHUNK,), jnp.float32),
            pltpu.VMEM_SHARED((N,), jnp.float32),
            pltpu.SemaphoreType.DMA,
        ],
    )
    def deg_kernel(dstm_hbm, out_hbm, dstidx, valbuf, zbuf, acc, sem):
        c = lax.axis_index("c")
        s = lax.axis_index("s")
        wid = c * NS + s

        ones16 = jnp.ones((16,), jnp.float32)
        zero16 = jnp.zeros((16,), jnp.float32)

        def fill(i, _):
            valbuf[pl.ds(i * 16, 16)] = ones16
            return 0
        lax.fori_loop(0, CHUNK // 16, fill, 0)

        def zfill(i, _):
            zbuf[pl.ds(i * 16, 16)] = zero16
            return 0
        lax.fori_loop(0, ZCHUNK // 16, zfill, 0)

        pltpu.sync_copy(dstm_hbm.at[wid], dstidx)

        def acc_zero(j, _):
            k = s * 8 + j

            @pl.when(k < RCHUNKS)
            def _():
                pltpu.sync_copy(zbuf, acc.at[pl.ds(k * ZCHUNK, ZCHUNK)])
            return 0
        lax.fori_loop(0, 8, acc_zero, 0)
        plsc.subcore_barrier()

        def fire(g, _):
            pltpu.async_copy(valbuf, acc.at[dstidx.at[g]], sem, add=True)
            return 0
        lax.fori_loop(0, NFULL, fire, 0)

        def drain(g, _):
            pltpu.make_async_copy(valbuf, acc.at[dstidx.at[0]], sem).wait()
            return 0
        lax.fori_loop(0, NFULL, drain, 0)

        plsc.subcore_barrier()

        obase = c * N

        def dump(j, _):
            k = s * 8 + j

            @pl.when(k < RCHUNKS)
            def _():
                pltpu.sync_copy(acc.at[pl.ds(k * ZCHUNK, ZCHUNK)], zbuf)
                pltpu.sync_copy(zbuf, out_hbm.at[pl.ds(obase + k * ZCHUNK, ZCHUNK)])
            return 0
        lax.fori_loop(0, 8, dump, 0)

    return deg_kernel(dstm)


def _sc_scatter(table, srcm, dstmA, dstmB):
    @functools.partial(
        pl.kernel,
        out_type=jax.ShapeDtypeStruct((NC, N, H), jnp.float32),
        mesh=_SC_MESH,
        scratch_types=[
            pltpu.VMEM((EPW,), jnp.int32),
            pltpu.VMEM((PH0, CHUNK), jnp.int32),
            [pltpu.VMEM((CHUNK, H), jnp.float32)] * NBUF,
            pltpu.VMEM_SHARED((N, H), jnp.float32),
            [pltpu.SemaphoreType.DMA] * NBUF,
            [pltpu.SemaphoreType.DMA] * NBUF,
        ],
    )
    def scat_kernel(table_hbm, srcm_hbm, dstmA_hbm, dstmB_hbm,
                    out_hbm, srcidx, dstidx, rows, acc, gsems, ssems):
        c = lax.axis_index("c")
        s = lax.axis_index("s")
        wid = c * NS + s

        zero16 = jnp.zeros((16,), jnp.float32)

        pltpu.sync_copy(srcm_hbm.at[pl.ds(wid * EPW, EPW)], srcidx)
        pltpu.sync_copy(dstmA_hbm.at[wid], dstidx)

        def zrow(r, _):
            def zcol(cc, _):
                rows[0][r, pl.ds(cc * 16, 16)] = zero16
                return 0
            lax.fori_loop(0, H // 16, zcol, 0)
            return 0
        lax.fori_loop(0, ZCHUNK, zrow, 0)

        def acc_zero(j, _):
            k = s * 8 + j

            @pl.when(k < RCHUNKS)
            def _():
                pltpu.sync_copy(rows[0].at[pl.ds(0, ZCHUNK), :],
                                acc.at[pl.ds(k * ZCHUNK, ZCHUNK), :])
            return 0
        lax.fori_loop(0, 8, acc_zero, 0)
        plsc.subcore_barrier()

        def gather(g, b):
            pltpu.async_copy(table_hbm.at[srcidx.at[pl.ds(g * CHUNK, CHUNK)]],
                             rows[b], gsems[b])

        def gather_wait(b):
            pltpu.make_async_copy(
                table_hbm.at[srcidx.at[pl.ds(0, CHUNK)]], rows[b],
                gsems[b]).wait()

        def scatter(g, b):
            pltpu.async_copy(rows[b], acc.at[dstidx.at[g]], ssems[b],
                             add=True)

        def scatter_wait(b):
            pltpu.make_async_copy(rows[b], acc.at[dstidx.at[0]],
                                  ssems[b]).wait()

        for b in range(NBUF):
            gather(b, b)

        def round0(r, _):
            for b in range(NBUF):
                gather_wait(b)
                scatter(NBUF * r + b, b)
            for b in range(NBUF):
                scatter_wait(b)
                g = NBUF * r + NBUF + b
                gather(g, b)
            return 0
        lax.fori_loop(0, NR0, round0, 0)

        pltpu.sync_copy(dstmB_hbm.at[wid], dstidx.at[pl.ds(0, PH1), :])

        def round1(r, _):
            for b in range(NBUF):
                gather_wait(b)
                scatter(NBUF * r + b, b)
            for b in range(NBUF):
                scatter_wait(b)
                g = PH0 + NBUF * r + NBUF + b

                @pl.when(g < NFULL)
                def _():
                    gather(g, b)
            return 0
        lax.fori_loop(0, NR1, round1, 0)

        for i in range(NLEFT):
            gather_wait(i)
            scatter(NR1 * NBUF + i, i)
            scatter_wait(i)
        plsc.subcore_barrier()

        def dump(j, _):
            k = s * 8 + j

            @pl.when(k < RCHUNKS)
            def _():
                pltpu.sync_copy(acc.at[pl.ds(k * ZCHUNK, ZCHUNK), :],
                                rows[0].at[pl.ds(0, ZCHUNK), :])
                pltpu.sync_copy(rows[0].at[pl.ds(0, ZCHUNK), :],
                                out_hbm.at[c, pl.ds(k * ZCHUNK, ZCHUNK), :])
            return 0
        lax.fori_loop(0, 8, dump, 0)

    return scat_kernel(table, srcm, dstmA, dstmB)


_BLK = 1000
_NBLK = N // _BLK


def _tc_prescale(x, W1, degp):
    def body(x_ref, w_ref, dp_ref, hw_ref, dinv_ref):
        deg = 1.0 + dp_ref[0] + dp_ref[1]
        dinv = lax.rsqrt(deg)
        dinv_ref[...] = dinv
        hw_ref[...] = jnp.dot(x_ref[...], w_ref[...],
                              preferred_element_type=jnp.float32) * dinv

    return pl.pallas_call(
        body,
        grid=(_NBLK,),
        in_specs=[
            pl.BlockSpec((_BLK, D), lambda i: (i, 0)),
            pl.BlockSpec((D, H), lambda i: (0, 0)),
            pl.BlockSpec((NC, _BLK, 1), lambda i: (0, i, 0)),
        ],
        out_specs=[
            pl.BlockSpec((_BLK, H), lambda i: (i, 0)),
            pl.BlockSpec((_BLK, 1), lambda i: (i, 0)),
        ],
        out_shape=[
            jax.ShapeDtypeStruct((N, H), jnp.float32),
            jax.ShapeDtypeStruct((N, 1), jnp.float32),
        ],
    )(x, W1, degp)


def _tc_layer_mid(Sp, hwp, dinv, b, W2):
    def body(s_ref, hw_ref, dinv_ref, b_ref, w_ref, out_ref):
        dinv = dinv_ref[...]
        h = s_ref[0] + s_ref[1] + hw_ref[...]
        h = jnp.maximum(dinv * h + b_ref[...], 0.0)
        out_ref[...] = jnp.dot(h, w_ref[...],
                               preferred_element_type=jnp.float32) * dinv

    return pl.pallas_call(
        body,
        grid=(_NBLK,),
        in_specs=[
            pl.BlockSpec((NC, _BLK, H), lambda i: (0, i, 0)),
            pl.BlockSpec((_BLK, H), lambda i: (i, 0)),
            pl.BlockSpec((_BLK, 1), lambda i: (i, 0)),
            pl.BlockSpec((1, H), lambda i: (0, 0)),
            pl.BlockSpec((H, H), lambda i: (0, 0)),
        ],
        out_specs=pl.BlockSpec((_BLK, H), lambda i: (i, 0)),
        out_shape=jax.ShapeDtypeStruct((N, H), jnp.float32),
    )(Sp, hwp, dinv, b, W2)


def _tc_finish_pool(Sp, hwp, dinv, b, batch2d):
    def body(s_ref, hw_ref, dinv_ref, b_ref, bat_ref, out_ref, cnt_ref):
        i = pl.program_id(0)
        dinv = dinv_ref[...]
        h = s_ref[0] + s_ref[1] + hw_ref[...]
        h = jnp.maximum(dinv * h + b_ref[...], 0.0)

        gids = lax.broadcasted_iota(jnp.int32, (_BLK, G), 1)
        onehot = (bat_ref[...] == gids).astype(jnp.float32)
        part = lax.dot_general(onehot, h, (((0,), (0,)), ((), ())),
                               preferred_element_type=jnp.float32)
        pcnt = lax.dot_general(onehot, jnp.ones((_BLK, 1), jnp.float32),
                               (((0,), (0,)), ((), ())),
                               preferred_element_type=jnp.float32)

        @pl.when(i == 0)
        def _():
            out_ref[...] = jnp.zeros_like(out_ref)
            cnt_ref[...] = jnp.zeros_like(cnt_ref)

        out_ref[...] += part
        cnt_ref[...] += pcnt

        @pl.when(i == _NBLK - 1)
        def _():
            out_ref[...] = out_ref[...] / jnp.maximum(cnt_ref[...], 1.0)

    return pl.pallas_call(
        body,
        grid=(_NBLK,),
        in_specs=[
            pl.BlockSpec((NC, _BLK, H), lambda i: (0, i, 0)),
            pl.BlockSpec((_BLK, H), lambda i: (i, 0)),
            pl.BlockSpec((_BLK, 1), lambda i: (i, 0)),
            pl.BlockSpec((1, H), lambda i: (0, 0)),
            pl.BlockSpec((_BLK, 1), lambda i: (i, 0)),
        ],
        out_specs=pl.BlockSpec((G, H), lambda i: (0, 0)),
        out_shape=jax.ShapeDtypeStruct((G, H), jnp.float32),
        scratch_shapes=[pltpu.VMEM((G, 1), jnp.float32)],
    )(Sp, hwp, dinv, b, batch2d)


def kernel(x, edge_index, batch, W1, b1, W2, b2):
    srcf = edge_index[0]
    dstm = edge_index[1].reshape(NW, NFULL, CHUNK)
    dstmA = dstm[:, :PH0, :]
    dstmB = dstm[:, PH0:, :]

    degp = _sc_degree(dstm)
    degp3 = degp.reshape(NC, N, 1)

    hw1p, dinv = _tc_prescale(x, W1, degp3)
    S1 = _sc_scatter(hw1p, srcf, dstmA, dstmB)
    hw2p = _tc_layer_mid(S1, hw1p, dinv, b1.reshape(1, H), W2)
    S2 = _sc_scatter(hw2p, srcf, dstmA, dstmB)
    g = _tc_finish_pool(S2, hw2p, dinv, b2.reshape(1, H),
                        batch.reshape(N, 1))
    return g
